# Initial kernel scaffold; baseline (speedup 1.0000x reference)
import jax
import jax.numpy as jnp
from jax.experimental import pallas as pl


def kernel(x, edge_index, W1, b1, W2, b2, Wout, bout):
    raise NotImplementedError("write your pallas kernel here")


# speedup vs baseline: 6.2017x; 6.2017x over previous
import functools

import jax
import jax.numpy as jnp
from jax import lax
from jax.experimental import pallas as pl
from jax.experimental.pallas import tpu as pltpu
from jax.experimental.pallas import tpu_sc as plsc

N = 10000
D = 256
DH = 128
NP = 10240
E = 160000
EP = 163840
NC = 2
NS = 16
RPT = NP // NS
R = 1000
GRID = N // R

_MESH = dict(core_axis_name="c", subcore_axis_name="s", num_cores=NC,
             num_subcores=NS)


def _hist_body(dst_hbm, o_hbm, z_hbm, out_hbm, idx_v, ones_v, hist_sh, sem):
    c = lax.axis_index("c")
    s = lax.axis_index("s")
    chunk = (c * NS + s) * (EP // 128 // (NC * NS))
    pltpu.sync_copy(dst_hbm.at[pl.ds(chunk, 40)], idx_v)
    pltpu.sync_copy(o_hbm, ones_v)
    pltpu.sync_copy(z_hbm, hist_sh.at[pl.ds(s * RPT, RPT)])
    plsc.subcore_barrier()

    def body(j, carry):
        pltpu.sync_copy(ones_v, hist_sh.at[idx_v.at[j]], add=True)
        return carry

    lax.fori_loop(0, 40, body, 0)
    plsc.subcore_barrier()
    pltpu.sync_copy(hist_sh.at[pl.ds(s * RPT, RPT)],
                    out_hbm.at[pl.ds(c * NP + s * RPT, RPT)])


@functools.cache
def _hist_kernel():
    return functools.partial(
        pl.kernel,
        out_type=jax.ShapeDtypeStruct((NC * NP, DH), jnp.float32),
        mesh=plsc.VectorSubcoreMesh(**_MESH),
        scratch_types=[
            pltpu.VMEM((40, 128), jnp.int32),
            pltpu.VMEM((128, DH), jnp.float32),
            pltpu.VMEM_SHARED((NP, DH), jnp.float32),
            pltpu.SemaphoreType.DMA,
        ],
    )(_hist_body)


def _scatter_body(y_hbm, si_hbm, di_hbm, z_hbm, out_hbm,
                  sidx_v, didx_v, rows_v, agg_sh, sem):
    c = lax.axis_index("c")
    s = lax.axis_index("s")
    nrow = EP // 128 // NS
    pltpu.sync_copy(si_hbm.at[pl.ds(c * (EP // 128) + s * nrow, nrow)], sidx_v)
    pltpu.sync_copy(di_hbm.at[pl.ds(s * nrow, nrow)], didx_v)
    pltpu.sync_copy(z_hbm, agg_sh.at[pl.ds(s * RPT, RPT)])
    plsc.subcore_barrier()

    def body(j, carry):
        pltpu.async_copy(y_hbm.at[sidx_v.at[j]], rows_v, sem).wait()
        pltpu.sync_copy(rows_v, agg_sh.at[didx_v.at[j]], add=True)
        return carry

    lax.fori_loop(0, nrow, body, 0)
    plsc.subcore_barrier()
    pltpu.sync_copy(agg_sh.at[pl.ds(s * RPT, RPT)],
                    out_hbm.at[pl.ds(c * NP + s * RPT, RPT)])


@functools.cache
def _scatter_kernel():
    return functools.partial(
        pl.kernel,
        out_type=jax.ShapeDtypeStruct((NC * NP, DH), jnp.float32),
        mesh=plsc.VectorSubcoreMesh(**_MESH),
        scratch_types=[
            pltpu.VMEM((EP // 128 // NS, 128), jnp.int32),
            pltpu.VMEM((EP // 128 // NS, 128), jnp.int32),
            pltpu.VMEM((128, DH), jnp.float32),
            pltpu.VMEM_SHARED((NP, DH), jnp.float32),
            pltpu.SemaphoreType.DMA,
        ],
    )(_scatter_body)


def _dinv(h):
    return lax.rsqrt(h[:, 0:1] + h[:, 1:2] + 1.0)


def _dot(a, b):
    return jnp.dot(a, b, preferred_element_type=jnp.float32,
                   precision=lax.Precision.HIGHEST)


def _tc_prep_body(x_ref, w_ref, h_ref, y_ref):
    dinv = _dinv(h_ref[...])
    y = _dot(x_ref[...], w_ref[...]) * dinv
    y_ref[0] = y[:, :DH]
    y_ref[1] = y[:, DH:]


def _tc_mid_body(agg_ref, y_ref, h_ref, w_ref, b_ref, out_ref):
    dinv = _dinv(h_ref[...])
    aggc = jnp.concatenate([agg_ref[0], agg_ref[1]], axis=1)
    yc = jnp.concatenate([y_ref[0], y_ref[1]], axis=1)
    hid = jnp.maximum(dinv * (aggc + yc) + b_ref[...], 0.0)
    y2 = _dot(hid, w_ref[...]) * dinv
    out_ref[0] = y2[:, :DH]
    out_ref[1] = y2[:, DH:]


def _tc_fin_body(agg_ref, y_ref, h_ref, b_ref, wo_ref, bo_ref, out_ref):
    dinv = _dinv(h_ref[...])
    aggc = jnp.concatenate([agg_ref[0], agg_ref[1]], axis=1)
    yc = jnp.concatenate([y_ref[0], y_ref[1]], axis=1)
    hid = jnp.maximum(dinv * (aggc + yc) + b_ref[...], 0.0)
    logits = _dot(hid, wo_ref[...]) + bo_ref[...]
    m = jnp.max(logits, axis=1, keepdims=True)
    lse = jnp.log(jnp.sum(jnp.exp(logits - m), axis=1, keepdims=True)) + m
    out_ref[...] = logits - lse


def _spec_rows(shape):
    return pl.BlockSpec((R,) + shape[1:], lambda i: (i,) + (0,) * len(shape[1:]))


def _spec_full(shape):
    return pl.BlockSpec(shape, lambda i: (0,) * len(shape))


_SPLIT_SPEC = pl.BlockSpec((NC, R, DH), lambda i: (0, i, 0))


def _tc_prep(x, w1, hist_t):
    return pl.pallas_call(
        _tc_prep_body,
        grid=(GRID,),
        in_specs=[_spec_rows((N, D)), _spec_full((D, D)), _spec_rows((N, 2))],
        out_specs=_SPLIT_SPEC,
        out_shape=jax.ShapeDtypeStruct((NC, N, DH), jnp.float32),
    )(x, w1, hist_t)


def _tc_mid(agg, y, hist_t, w2, b1):
    return pl.pallas_call(
        _tc_mid_body,
        grid=(GRID,),
        in_specs=[_SPLIT_SPEC, _SPLIT_SPEC, _spec_rows((N, 2)),
                  _spec_full((D, D)), _spec_full((1, D))],
        out_specs=_SPLIT_SPEC,
        out_shape=jax.ShapeDtypeStruct((NC, N, DH), jnp.float32),
    )(agg, y, hist_t, w2, b1)


def _tc_fin(agg, y, hist_t, b2, wout, bout):
    return pl.pallas_call(
        _tc_fin_body,
        grid=(GRID,),
        in_specs=[_SPLIT_SPEC, _SPLIT_SPEC, _spec_rows((N, 2)),
                  _spec_full((1, D)),
                  _spec_full((D, 32)), _spec_full((1, 32))],
        out_specs=_spec_rows((N, 32)),
        out_shape=jax.ShapeDtypeStruct((N, 32), jnp.float32),
    )(agg, y, hist_t, b2, wout, bout)


def kernel(x, edge_index, W1, b1, W2, b2, Wout, bout):
    src = edge_index[0].astype(jnp.int32)
    dst = edge_index[1].astype(jnp.int32)
    pad = EP - E
    src_p = jnp.concatenate([src, jnp.zeros((pad,), jnp.int32)])
    dst_p = jnp.concatenate([dst, jnp.full((pad,), N, jnp.int32)])
    src2 = jnp.concatenate([src_p, src_p + N]).reshape(2 * EP // 128, 128)
    dst_r = dst_p.reshape(EP // 128, 128)

    o128 = jnp.ones((128, DH), jnp.float32)
    z128 = jnp.zeros((RPT, DH), jnp.float32)

    hist = _hist_kernel()(dst_r, o128, z128).reshape(NC, NP, DH)
    hist_t = hist[:, :N, 0].T

    y1 = _tc_prep(x, W1, hist_t)
    agg1 = _scatter_kernel()(y1.reshape(NC * N, DH), src2, dst_r, z128)
    agg1 = agg1.reshape(NC, NP, DH)

    y2 = _tc_mid(agg1, y1, hist_t, W2, b1.reshape(1, D))
    agg2 = _scatter_kernel()(y2.reshape(NC * N, DH), src2, dst_r, z128)
    agg2 = agg2.reshape(NC, NP, DH)

    return _tc_fin(agg2, y2, hist_t, b2.reshape(1, D),
                   Wout, bout.reshape(1, 32))


# speedup vs baseline: 6.3186x; 1.0188x over previous
import functools

import jax
import jax.numpy as jnp
from jax import lax
from jax.experimental import pallas as pl
from jax.experimental.pallas import tpu as pltpu
from jax.experimental.pallas import tpu_sc as plsc

N = 10000
D = 256
DH = 128
NP = 10240
E = 160000
EP = 163840
NC = 2
NS = 16
RPT = NP // NS
R = 1000
GRID = N // R

_MESH = dict(core_axis_name="c", subcore_axis_name="s", num_cores=NC,
             num_subcores=NS)


NH = NP // NC
NHP = NH + 128


def _hist_body(dst_hbm, o_hbm, z_hbm, out_hbm, idx_v, ones_v, hist_sh, sem):
    c = lax.axis_index("c")
    s = lax.axis_index("s")
    nrow = EP // 128 // NS
    pltpu.sync_copy(dst_hbm.at[pl.ds(c * (EP // 128) + s * nrow, nrow)],
                    idx_v)
    pltpu.sync_copy(o_hbm, ones_v)
    pltpu.sync_copy(z_hbm, hist_sh.at[pl.ds(s * (NHP // NS), NHP // NS)])
    plsc.subcore_barrier()

    @pl.loop(0, nrow)
    def _issue(j):
        @pl.when(j >= 8)
        def _():
            pltpu.make_async_copy(o_hbm, ones_v, sem).wait()
        pltpu.async_copy(ones_v, hist_sh.at[idx_v.at[j]], sem, add=True)

    @pl.loop(0, 8)
    def _drain(j):
        pltpu.make_async_copy(o_hbm, ones_v, sem).wait()
    plsc.subcore_barrier()
    pltpu.sync_copy(hist_sh.at[pl.ds(s * (NH // NS), NH // NS)],
                    out_hbm.at[pl.ds(c * NH + s * (NH // NS), NH // NS)])


@functools.cache
def _hist_kernel():
    return functools.partial(
        pl.kernel,
        out_type=jax.ShapeDtypeStruct((NP, DH), jnp.float32),
        mesh=plsc.VectorSubcoreMesh(**_MESH),
        scratch_types=[
            pltpu.VMEM((EP // 128 // NS, 128), jnp.int32),
            pltpu.VMEM((128, DH), jnp.float32),
            pltpu.VMEM_SHARED((NHP, DH), jnp.float32),
            pltpu.SemaphoreType.DMA,
        ],
    )(_hist_body)


def _scatter_body(y_hbm, si_hbm, di_hbm, z_hbm, out_hbm,
                  sidx_v, didx_v, row_v, agg_sh):
    c = lax.axis_index("c")
    s = lax.axis_index("s")
    nrow = EP // 128 // NS
    pltpu.sync_copy(si_hbm.at[pl.ds(c * (EP // 128) + s * nrow, nrow)], sidx_v)
    pltpu.sync_copy(di_hbm.at[pl.ds(s * nrow, nrow)], didx_v)
    pltpu.sync_copy(z_hbm, agg_sh.at[pl.ds(s * RPT, RPT)])
    plsc.subcore_barrier()

    @pl.loop(0, nrow)
    def _batch(j):
        pltpu.sync_copy(y_hbm.at[sidx_v.at[j]], row_v)
        pltpu.sync_copy(row_v, agg_sh.at[didx_v.at[j]], add=True)

    plsc.subcore_barrier()
    pltpu.sync_copy(agg_sh.at[pl.ds(s * RPT, RPT)],
                    out_hbm.at[pl.ds(c * NP + s * RPT, RPT)])


@functools.cache
def _scatter_kernel():
    return functools.partial(
        pl.kernel,
        out_type=jax.ShapeDtypeStruct((NC * NP, DH), jnp.float32),
        mesh=plsc.VectorSubcoreMesh(**_MESH),
        scratch_types=[
            pltpu.VMEM((EP // 128 // NS, 128), jnp.int32),
            pltpu.VMEM((EP // 128 // NS, 128), jnp.int32),
            pltpu.VMEM((128, DH), jnp.float32),
            pltpu.VMEM_SHARED((NP, DH), jnp.float32),
        ],
    )(_scatter_body)


def _dinv(h_ref):
    return lax.rsqrt(h_ref[:, 0:1] + 1.0)


def _dot(a, b):
    return jnp.dot(a, b, preferred_element_type=jnp.float32,
                   precision=lax.Precision.HIGHEST)


def _tc_prep_body(x_ref, w_ref, h_ref, y_ref):
    dinv = _dinv(h_ref)
    y = _dot(x_ref[...], w_ref[...]) * dinv
    y_ref[0] = y[:, :DH]
    y_ref[1] = y[:, DH:]


def _tc_mid_body(agg_ref, y_ref, h_ref, w_ref, b_ref, out_ref):
    dinv = _dinv(h_ref)
    aggc = jnp.concatenate([agg_ref[0], agg_ref[1]], axis=1)
    yc = jnp.concatenate([y_ref[0], y_ref[1]], axis=1)
    hid = jnp.maximum(dinv * (aggc + yc) + b_ref[...], 0.0)
    y2 = _dot(hid, w_ref[...]) * dinv
    out_ref[0] = y2[:, :DH]
    out_ref[1] = y2[:, DH:]


def _tc_fin_body(agg_ref, y_ref, h_ref, b_ref, wo_ref, bo_ref, out_ref):
    dinv = _dinv(h_ref)
    aggc = jnp.concatenate([agg_ref[0], agg_ref[1]], axis=1)
    yc = jnp.concatenate([y_ref[0], y_ref[1]], axis=1)
    hid = jnp.maximum(dinv * (aggc + yc) + b_ref[...], 0.0)
    logits = _dot(hid, wo_ref[...]) + bo_ref[...]
    m = jnp.max(logits, axis=1, keepdims=True)
    lse = jnp.log(jnp.sum(jnp.exp(logits - m), axis=1, keepdims=True)) + m
    out_ref[...] = logits - lse


def _spec_rows(shape):
    return pl.BlockSpec((R,) + shape[1:], lambda i: (i,) + (0,) * len(shape[1:]))


def _spec_full(shape):
    return pl.BlockSpec(shape, lambda i: (0,) * len(shape))


_SPLIT_SPEC = pl.BlockSpec((NC, R, DH), lambda i: (0, i, 0))
_HIST_SPEC = pl.BlockSpec((R, DH), lambda i: (i, 0))


def _tc_prep(x, w1, hist):
    return pl.pallas_call(
        _tc_prep_body,
        grid=(GRID,),
        in_specs=[_spec_rows((N, D)), _spec_full((D, D)), _HIST_SPEC],
        out_specs=_SPLIT_SPEC,
        out_shape=jax.ShapeDtypeStruct((NC, N, DH), jnp.float32),
    )(x, w1, hist)


def _tc_mid(agg, y, hist, w2, b1):
    return pl.pallas_call(
        _tc_mid_body,
        grid=(GRID,),
        in_specs=[_SPLIT_SPEC, _SPLIT_SPEC, _HIST_SPEC,
                  _spec_full((D, D)), _spec_full((1, D))],
        out_specs=_SPLIT_SPEC,
        out_shape=jax.ShapeDtypeStruct((NC, N, DH), jnp.float32),
    )(agg, y, hist, w2, b1)


def _tc_fin(agg, y, hist, b2, wout, bout):
    return pl.pallas_call(
        _tc_fin_body,
        grid=(GRID,),
        in_specs=[_SPLIT_SPEC, _SPLIT_SPEC, _HIST_SPEC,
                  _spec_full((1, D)),
                  _spec_full((D, 32)), _spec_full((1, 32))],
        out_specs=_spec_rows((N, 32)),
        out_shape=jax.ShapeDtypeStruct((N, 32), jnp.float32),
    )(agg, y, hist, b2, wout, bout)


def kernel(x, edge_index, W1, b1, W2, b2, Wout, bout):
    src = edge_index[0].astype(jnp.int32)
    dst = edge_index[1].astype(jnp.int32)
    pad = EP - E
    src_p = jnp.concatenate([src, jnp.zeros((pad,), jnp.int32)])
    dst_p = jnp.concatenate([dst, jnp.full((pad,), N, jnp.int32)])
    src2 = jnp.concatenate([src_p, src_p + N]).reshape(2 * EP // 128, 128)
    dst_r = dst_p.reshape(EP // 128, 128)

    o128 = jnp.ones((128, DH), jnp.float32)
    z128 = jnp.zeros((RPT, DH), jnp.float32)
    zh = jnp.zeros((NHP // NS, DH), jnp.float32)

    dsth = jnp.stack([jnp.where((dst_p >= c * NH) & (dst_p < (c + 1) * NH),
                                dst_p - c * NH, NH) for c in range(NC)])
    dsth_r = dsth.reshape(NC * EP // 128, 128)

    hist = _hist_kernel()(dsth_r, o128, zh)

    y1 = _tc_prep(x, W1, hist)
    agg1 = _scatter_kernel()(y1.reshape(NC * N, DH), src2, dst_r, z128)
    agg1 = agg1.reshape(NC, NP, DH)

    y2 = _tc_mid(agg1, y1, hist, W2, b1.reshape(1, D))
    agg2 = _scatter_kernel()(y2.reshape(NC * N, DH), src2, dst_r, z128)
    agg2 = agg2.reshape(NC, NP, DH)

    return _tc_fin(agg2, y2, hist, b2.reshape(1, D),
                   Wout, bout.reshape(1, 32))
